# baseline (speedup 1.0000x reference)
import functools

import jax
import jax.numpy as jnp
from jax import lax
from jax.experimental import pallas as pl
from jax.experimental.pallas import tpu as pltpu
from jax.experimental.pallas import tpu_sc as plsc

N_NODES = 10000
N_EDGES = 320000
D = 128
NC = 2
NS = 16
NW = NC * NS
GC = 128
EWT = N_EDGES // NW
NCHF = EWT // GC
TAIL = EWT - NCHF * GC
ZR = 40
RPT = 640


def _tc_node_transform(h, W1a, W1b):
    blk = 2000

    def body(h_ref, wa_ref, wb_ref, p_ref, q_ref):
        hb = h_ref[...]
        p_ref[...] = jnp.dot(hb, wa_ref[...], preferred_element_type=jnp.float32)
        q_ref[...] = jnp.dot(hb, wb_ref[...], preferred_element_type=jnp.float32)

    return pl.pallas_call(
        body,
        grid=(N_NODES // blk,),
        in_specs=[
            pl.BlockSpec((blk, D), lambda i: (i, 0)),
            pl.BlockSpec((D, D), lambda i: (0, 0)),
            pl.BlockSpec((D, D), lambda i: (0, 0)),
        ],
        out_specs=[pl.BlockSpec((blk, D), lambda i: (i, 0))] * 2,
        out_shape=[jax.ShapeDtypeStruct((N_NODES, D), jnp.float32)] * 2,
    )(h, W1a, W1b)


def _sc_gather_add(P, Q, rows3, cols3, ew, nch, tail):
    mesh = plsc.VectorSubcoreMesh(core_axis_name="c", subcore_axis_name="s")
    nsteps = nch + (1 if tail else 0)
    sizes = [GC] * nch + ([tail] if tail else [])

    @functools.partial(
        pl.kernel,
        out_type=jax.ShapeDtypeStruct((NW * ew, D), jnp.float32),
        mesh=mesh,
        scratch_types=[
            pltpu.VMEM((1, ew), jnp.int32),
            pltpu.VMEM((1, ew), jnp.int32),
            pltpu.VMEM((2, GC, D), jnp.float32),
            pltpu.VMEM((2, GC, D), jnp.float32),
            pltpu.VMEM((2, GC, D), jnp.float32),
        ] + [pltpu.SemaphoreType.DMA] * 6,
    )
    def k(p_hbm, q_hbm, r_hbm, c_hbm, t_hbm, idxa, idxb, bufa, bufb, obuf,
          sema0, sema1, semb0, semb1, wsem0, wsem1):
        sema = (sema0, sema1)
        semb = (semb0, semb1)
        wsem = (wsem0, wsem1)
        wid = lax.axis_index("s") * NC + lax.axis_index("c")
        e0 = wid * ew
        pltpu.sync_copy(r_hbm.at[wid], idxa)
        pltpu.sync_copy(c_hbm.at[wid], idxb)

        def _gops(j, b):
            n = sizes[j]
            parts = [(0, n)] if n < GC else [(0, GC // 2), (GC // 2, GC // 2)]
            for (o, m) in parts:
                ia = idxa.at[0, pl.ds(j * GC + o, m)]
                ib = idxb.at[0, pl.ds(j * GC + o, m)]
                yield (p_hbm.at[ia], bufa.at[b, pl.ds(o, m)], sema[b])
                yield (q_hbm.at[ib], bufb.at[b, pl.ds(o, m)], semb[b])

        def issue(j, b):
            for (src, dst, sem) in _gops(j, b):
                pltpu.async_copy(src, dst, sem)

        def wait_gather(j, b):
            for (src, dst, sem) in _gops(j, b):
                pltpu.make_async_copy(src, dst, sem).wait()

        def wait_write(j, b):
            n = sizes[j]
            pltpu.make_async_copy(obuf.at[b, pl.ds(0, n)],
                                  t_hbm.at[pl.ds(e0 + j * GC, n)],
                                  wsem[b]).wait()

        issue(0, 0)
        issue(1, 1)
        for j in range(nsteps):
            b = j % 2
            n = sizes[j]
            if j >= 2:
                wait_write(j - 2, b)
            wait_gather(j, b)

            @pl.loop(0, n)
            def _row(r):
                for v in range(D // 16):
                    sl = pl.ds(v * 16, 16)
                    obuf[b, r, sl] = bufa[b, r, sl] + bufb[b, r, sl]

            if j + 2 < nsteps:
                issue(j + 2, b)
            pltpu.async_copy(obuf.at[b, pl.ds(0, n)],
                             t_hbm.at[pl.ds(e0 + j * GC, n)], wsem[b])
        wait_write(nsteps - 2, (nsteps - 2) % 2)
        wait_write(nsteps - 1, (nsteps - 1) % 2)

    return k(P, Q, rows3, cols3)


def _tc_mlp(T, ef, W1e, b1, W2, b2):
    blk = 2560
    ne = T.shape[0]
    F = ef.shape[1]

    def body(t_ref, e_ref, we_ref, b1_ref, w2_ref, b2_ref, o_ref):
        pre = t_ref[...] + jnp.dot(e_ref[...], we_ref[...],
                                   preferred_element_type=jnp.float32)
        hid = jnp.maximum(pre + b1_ref[...], 0.0)
        o_ref[...] = jnp.dot(hid, w2_ref[...],
                             preferred_element_type=jnp.float32) + b2_ref[...]

    return pl.pallas_call(
        body,
        grid=(ne // blk,),
        in_specs=[
            pl.BlockSpec((blk, D), lambda i: (i, 0)),
            pl.BlockSpec((blk, F), lambda i: (i, 0)),
            pl.BlockSpec((F, D), lambda i: (0, 0)),
            pl.BlockSpec((1, D), lambda i: (0, 0)),
            pl.BlockSpec((D, D), lambda i: (0, 0)),
            pl.BlockSpec((1, D), lambda i: (0, 0)),
        ],
        out_specs=pl.BlockSpec((blk, D), lambda i: (i, 0)),
        out_shape=jax.ShapeDtypeStruct((ne, D), jnp.float32),
    )(T, ef, W1e, b1, W2, b2)


def _sc_scatter_add(M, idxall):
    mesh = plsc.VectorSubcoreMesh(core_axis_name="c", subcore_axis_name="s")

    @functools.partial(
        pl.kernel,
        out_type=jax.ShapeDtypeStruct((NC * N_NODES, D), jnp.float32),
        mesh=mesh,
        scratch_types=[
            pltpu.VMEM((NCHF + 1, GC), jnp.int32),
            pltpu.VMEM((2, GC, D), jnp.float32),
            pltpu.VMEM((ZR, D), jnp.float32),
            pltpu.VMEM_SHARED((N_NODES, D), jnp.float32),
            pltpu.SemaphoreType.DMA,
            pltpu.SemaphoreType.DMA,
        ],
    )
    def k(m_hbm, r_hbm, out_hbm, idxv, msgv, zbuf, hacc, rsem0, rsem1):
        rsem = (rsem0, rsem1)
        cid = lax.axis_index("c")
        sid = lax.axis_index("s")
        wid = sid * NC + cid

        @pl.loop(0, ZR)
        def _z(r):
            for v in range(D // 16):
                zbuf[r, pl.ds(v * 16, 16)] = jnp.zeros((16,), jnp.float32)

        r_base = sid * RPT
        n_cp = jnp.where(sid == NS - 1, (N_NODES - (NS - 1) * RPT) // ZR,
                         RPT // ZR)

        @pl.loop(0, n_cp)
        def _zc(kk):
            pltpu.sync_copy(zbuf, hacc.at[pl.ds(r_base + kk * ZR, ZR)])

        plsc.subcore_barrier()

        pltpu.sync_copy(r_hbm.at[wid], idxv)

        steps = ([(m_hbm, wid * EWT, j, idxv.at[j], GC) for j in range(NCHF)]
                 + [(m_hbm, wid * EWT, NCHF, idxv.at[NCHF, pl.ds(0, TAIL)],
                     TAIL)])

        def issue_read(s, b):
            src, base, j, _, n = s
            pltpu.async_copy(src.at[pl.ds(base + j * GC, n)],
                             msgv.at[b, pl.ds(0, n)], rsem[b])

        def wait_read(s, b):
            src, base, j, _, n = s
            pltpu.make_async_copy(src.at[pl.ds(base + j * GC, n)],
                                  msgv.at[b, pl.ds(0, n)], rsem[b]).wait()

        issue_read(steps[0], 0)
        for i, s in enumerate(steps):
            b = i % 2
            if i + 1 < len(steps):
                issue_read(steps[i + 1], 1 - b)
            wait_read(s, b)
            pltpu.sync_copy(msgv.at[b, pl.ds(0, s[4])], hacc.at[s[3]],
                            add=True)

        plsc.subcore_barrier()

        @pl.loop(0, n_cp)
        def _wb(kk):
            r0 = r_base + kk * ZR
            pltpu.sync_copy(hacc.at[pl.ds(r0, ZR)],
                            out_hbm.at[pl.ds(cid * N_NODES + r0, ZR)])

    return k(M, idxall)


def _tc_combine(S, delta_row):
    blk = 2000

    def body(a_ref, b_ref, dl_ref, o_ref):
        o_ref[...] = a_ref[...] + b_ref[...] + dl_ref[...]

    nblk = N_NODES // blk
    return pl.pallas_call(
        body,
        grid=(nblk,),
        in_specs=[
            pl.BlockSpec((blk, D), lambda i: (i, 0)),
            pl.BlockSpec((blk, D), lambda i, n=nblk: (i + n, 0)),
            pl.BlockSpec((1, D), lambda i: (0, 0)),
        ],
        out_specs=pl.BlockSpec((blk, D), lambda i: (i, 0)),
        out_shape=jax.ShapeDtypeStruct((N_NODES, D), jnp.float32),
    )(S, S, delta_row)


def kernel(h, edge_index, edge_features, n, W1, b1, W2, b2):
    rows = edge_index[0].astype(jnp.int32)
    cols = edge_index[1].astype(jnp.int32)

    W1a = W1[:D]
    W1b = W1[D:2 * D]
    W1e = W1[2 * D:]
    b1r = b1.reshape(1, D)
    b2r = b2.reshape(1, D)

    P, Q = _tc_node_transform(h, W1a, W1b)

    T = _sc_gather_add(P, Q, rows.reshape(NW, 1, EWT),
                       cols.reshape(NW, 1, EWT), EWT, NCHF, TAIL)
    M = _tc_mlp(T, edge_features, W1e, b1r, W2, b2r)

    rw = rows.reshape(NW, EWT)
    rtail = jnp.pad(rw[:, NCHF * GC:], ((0, 0), (0, GC - TAIL)))
    idxall = jnp.concatenate(
        [rw[:, :NCHF * GC].reshape(NW, NCHF, GC),
         rtail.reshape(NW, 1, GC)], axis=1)
    S = _sc_scatter_add(M, idxall)

    delta = (jnp.asarray(n) - N_NODES).astype(jnp.float32)
    delta_row = jnp.full((1, D), delta, dtype=jnp.float32)
    return _tc_combine(S, delta_row)


# speedup vs baseline: 1.0013x; 1.0013x over previous
import functools

import jax
import jax.numpy as jnp
from jax import lax
from jax.experimental import pallas as pl
from jax.experimental.pallas import tpu as pltpu
from jax.experimental.pallas import tpu_sc as plsc

N_NODES = 10000
N_EDGES = 320000
D = 128
NC = 2
NS = 16
NW = NC * NS
GC = 128
EWT = N_EDGES // NW
NCHF = EWT // GC
TAIL = EWT - NCHF * GC
ZR = 40
RPT = 640


def _tc_node_transform(h, W1a, W1b):
    blk = 2000

    def body(h_ref, wa_ref, wb_ref, p_ref, q_ref):
        hb = h_ref[...]
        p_ref[...] = jnp.dot(hb, wa_ref[...], preferred_element_type=jnp.float32)
        q_ref[...] = jnp.dot(hb, wb_ref[...], preferred_element_type=jnp.float32)

    return pl.pallas_call(
        body,
        grid=(N_NODES // blk,),
        in_specs=[
            pl.BlockSpec((blk, D), lambda i: (i, 0)),
            pl.BlockSpec((D, D), lambda i: (0, 0)),
            pl.BlockSpec((D, D), lambda i: (0, 0)),
        ],
        out_specs=[pl.BlockSpec((blk, D), lambda i: (i, 0))] * 2,
        out_shape=[jax.ShapeDtypeStruct((N_NODES, D), jnp.float32)] * 2,
    )(h, W1a, W1b)


def _sc_gather_add(P, Q, rows3, cols3, ew, nch, tail):
    mesh = plsc.VectorSubcoreMesh(core_axis_name="c", subcore_axis_name="s")
    nsteps = nch + (1 if tail else 0)
    sizes = [GC] * nch + ([tail] if tail else [])

    @functools.partial(
        pl.kernel,
        out_type=jax.ShapeDtypeStruct((NW * ew, D), jnp.float32),
        mesh=mesh,
        scratch_types=[
            pltpu.VMEM((1, ew), jnp.int32),
            pltpu.VMEM((1, ew), jnp.int32),
            pltpu.VMEM((2, GC, D), jnp.float32),
            pltpu.VMEM((2, GC, D), jnp.float32),
            pltpu.VMEM((2, GC, D), jnp.float32),
        ] + [pltpu.SemaphoreType.DMA] * 6,
    )
    def k(p_hbm, q_hbm, r_hbm, c_hbm, t_hbm, idxa, idxb, bufa, bufb, obuf,
          sema0, sema1, semb0, semb1, wsem0, wsem1):
        sema = (sema0, sema1)
        semb = (semb0, semb1)
        wsem = (wsem0, wsem1)
        wid = lax.axis_index("s") * NC + lax.axis_index("c")
        e0 = wid * ew
        pltpu.sync_copy(r_hbm.at[wid], idxa)
        pltpu.sync_copy(c_hbm.at[wid], idxb)

        def issue(j, b):
            n = sizes[j]
            ia = idxa.at[0, pl.ds(j * GC, n)]
            ib = idxb.at[0, pl.ds(j * GC, n)]
            pltpu.async_copy(p_hbm.at[ia], bufa.at[b, pl.ds(0, n)], sema[b])
            pltpu.async_copy(q_hbm.at[ib], bufb.at[b, pl.ds(0, n)], semb[b])

        def wait_gather(j, b):
            n = sizes[j]
            ia = idxa.at[0, pl.ds(j * GC, n)]
            ib = idxb.at[0, pl.ds(j * GC, n)]
            pltpu.make_async_copy(p_hbm.at[ia], bufa.at[b, pl.ds(0, n)],
                                  sema[b]).wait()
            pltpu.make_async_copy(q_hbm.at[ib], bufb.at[b, pl.ds(0, n)],
                                  semb[b]).wait()

        def wait_write(j, b):
            n = sizes[j]
            pltpu.make_async_copy(obuf.at[b, pl.ds(0, n)],
                                  t_hbm.at[pl.ds(e0 + j * GC, n)],
                                  wsem[b]).wait()

        issue(0, 0)
        issue(1, 1)
        for j in range(nsteps):
            b = j % 2
            n = sizes[j]
            if j >= 2:
                wait_write(j - 2, b)
            wait_gather(j, b)

            @pl.loop(0, n)
            def _row(r):
                for v in range(D // 16):
                    sl = pl.ds(v * 16, 16)
                    obuf[b, r, sl] = bufa[b, r, sl] + bufb[b, r, sl]

            if j + 2 < nsteps:
                issue(j + 2, b)
            pltpu.async_copy(obuf.at[b, pl.ds(0, n)],
                             t_hbm.at[pl.ds(e0 + j * GC, n)], wsem[b])
        wait_write(nsteps - 2, (nsteps - 2) % 2)
        wait_write(nsteps - 1, (nsteps - 1) % 2)

    return k(P, Q, rows3, cols3)


def _tc_mlp(T, ef, W1e, b1, W2, b2):
    blk = 2560
    ne = T.shape[0]
    F = ef.shape[1]

    def body(t_ref, e_ref, we_ref, b1_ref, w2_ref, b2_ref, o_ref):
        pre = t_ref[...] + jnp.dot(e_ref[...], we_ref[...],
                                   preferred_element_type=jnp.float32)
        hid = jnp.maximum(pre + b1_ref[...], 0.0)
        o_ref[...] = jnp.dot(hid, w2_ref[...],
                             preferred_element_type=jnp.float32) + b2_ref[...]

    return pl.pallas_call(
        body,
        grid=(ne // blk,),
        in_specs=[
            pl.BlockSpec((blk, D), lambda i: (i, 0)),
            pl.BlockSpec((blk, F), lambda i: (i, 0)),
            pl.BlockSpec((F, D), lambda i: (0, 0)),
            pl.BlockSpec((1, D), lambda i: (0, 0)),
            pl.BlockSpec((D, D), lambda i: (0, 0)),
            pl.BlockSpec((1, D), lambda i: (0, 0)),
        ],
        out_specs=pl.BlockSpec((blk, D), lambda i: (i, 0)),
        out_shape=jax.ShapeDtypeStruct((ne, D), jnp.float32),
    )(T, ef, W1e, b1, W2, b2)


def _sc_scatter_add(M, idxall):
    mesh = plsc.VectorSubcoreMesh(core_axis_name="c", subcore_axis_name="s")

    @functools.partial(
        pl.kernel,
        out_type=jax.ShapeDtypeStruct((NC * N_NODES, D), jnp.float32),
        mesh=mesh,
        scratch_types=[
            pltpu.VMEM((NCHF + 1, GC), jnp.int32),
            pltpu.VMEM((2, GC, D), jnp.float32),
            pltpu.VMEM((ZR, D), jnp.float32),
            pltpu.VMEM_SHARED((N_NODES, D), jnp.float32),
            pltpu.SemaphoreType.DMA,
            pltpu.SemaphoreType.DMA,
        ],
    )
    def k(m_hbm, r_hbm, out_hbm, idxv, msgv, zbuf, hacc, rsem0, rsem1):
        rsem = (rsem0, rsem1)
        cid = lax.axis_index("c")
        sid = lax.axis_index("s")
        wid = sid * NC + cid

        @pl.loop(0, ZR)
        def _z(r):
            for v in range(D // 16):
                zbuf[r, pl.ds(v * 16, 16)] = jnp.zeros((16,), jnp.float32)

        r_base = sid * RPT
        n_cp = jnp.where(sid == NS - 1, (N_NODES - (NS - 1) * RPT) // ZR,
                         RPT // ZR)

        @pl.loop(0, n_cp)
        def _zc(kk):
            pltpu.sync_copy(zbuf, hacc.at[pl.ds(r_base + kk * ZR, ZR)])

        plsc.subcore_barrier()

        pltpu.sync_copy(r_hbm.at[wid], idxv)

        steps = ([(m_hbm, wid * EWT, j, idxv.at[j], GC) for j in range(NCHF)]
                 + [(m_hbm, wid * EWT, NCHF, idxv.at[NCHF, pl.ds(0, TAIL)],
                     TAIL)])

        def issue_read(s, b):
            src, base, j, _, n = s
            pltpu.async_copy(src.at[pl.ds(base + j * GC, n)],
                             msgv.at[b, pl.ds(0, n)], rsem[b])

        def wait_read(s, b):
            src, base, j, _, n = s
            pltpu.make_async_copy(src.at[pl.ds(base + j * GC, n)],
                                  msgv.at[b, pl.ds(0, n)], rsem[b]).wait()

        issue_read(steps[0], 0)
        for i, s in enumerate(steps):
            b = i % 2
            if i + 1 < len(steps):
                issue_read(steps[i + 1], 1 - b)
            wait_read(s, b)
            pltpu.sync_copy(msgv.at[b, pl.ds(0, s[4])], hacc.at[s[3]],
                            add=True)

        plsc.subcore_barrier()

        @pl.loop(0, n_cp)
        def _wb(kk):
            r0 = r_base + kk * ZR
            pltpu.sync_copy(hacc.at[pl.ds(r0, ZR)],
                            out_hbm.at[pl.ds(cid * N_NODES + r0, ZR)])

    return k(M, idxall)


def _tc_combine(S, delta_row):
    blk = 2000

    def body(a_ref, b_ref, dl_ref, o_ref):
        o_ref[...] = a_ref[...] + b_ref[...] + dl_ref[...]

    nblk = N_NODES // blk
    return pl.pallas_call(
        body,
        grid=(nblk,),
        in_specs=[
            pl.BlockSpec((blk, D), lambda i: (i, 0)),
            pl.BlockSpec((blk, D), lambda i, n=nblk: (i + n, 0)),
            pl.BlockSpec((1, D), lambda i: (0, 0)),
        ],
        out_specs=pl.BlockSpec((blk, D), lambda i: (i, 0)),
        out_shape=jax.ShapeDtypeStruct((N_NODES, D), jnp.float32),
    )(S, S, delta_row)


def kernel(h, edge_index, edge_features, n, W1, b1, W2, b2):
    rows = edge_index[0].astype(jnp.int32)
    cols = edge_index[1].astype(jnp.int32)

    W1a = W1[:D]
    W1b = W1[D:2 * D]
    W1e = W1[2 * D:]
    b1r = b1.reshape(1, D)
    b2r = b2.reshape(1, D)

    P, Q = _tc_node_transform(h, W1a, W1b)

    T = _sc_gather_add(P, Q, rows.reshape(NW, 1, EWT),
                       cols.reshape(NW, 1, EWT), EWT, NCHF, TAIL)
    M = _tc_mlp(T, edge_features, W1e, b1r, W2, b2r)

    rw = rows.reshape(NW, EWT)
    rtail = jnp.pad(rw[:, NCHF * GC:], ((0, 0), (0, GC - TAIL)))
    idxall = jnp.concatenate(
        [rw[:, :NCHF * GC].reshape(NW, NCHF, GC),
         rtail.reshape(NW, 1, GC)], axis=1)
    S = _sc_scatter_add(M, idxall)

    delta = (jnp.asarray(n) - N_NODES).astype(jnp.float32)
    delta_row = jnp.full((1, D), delta, dtype=jnp.float32)
    return _tc_combine(S, delta_row)
